# baseline (speedup 1.0000x reference)
import jax
import jax.numpy as jnp
from jax import lax
from jax.experimental import pallas as pl
from jax.experimental.pallas import tpu as pltpu

_L = 256
_LB = 1024


def _ssd_kernel(x_ref, a_ref, b_ref, c_ref, init_ref, y_ref, st_ref):
    k = pl.program_id(1)
    h, p, n = st_ref.shape

    @pl.when(k == 0)
    def _():
        st_ref[...] = init_ref[0]

    row = lax.broadcasted_iota(jnp.int32, (_L, _L), 0)
    col = lax.broadcasted_iota(jnp.int32, (_L, _L), 1)
    ltri = row >= col

    for sub in range(_LB // _L):
        sl = slice(sub * _L, (sub + 1) * _L)
        at = jnp.transpose(a_ref[0, sl, :])

        csr = at
        sh = 1
        while sh < _L:
            csr = csr + jnp.concatenate(
                [jnp.zeros((h, sh), jnp.float32), csr[:, :_L - sh]], axis=1)
            sh *= 2

        cst = jnp.transpose(csr)
        epos_t = jnp.exp(cst)
        eneg_t = jnp.exp(-cst)
        elast = jnp.exp(csr[:, _L - 1:_L])

        for hi in range(h):
            x = x_ref[0, sl, hi * p:(hi + 1) * p]
            b = b_ref[0, sl, hi * n:(hi + 1) * n]
            c = c_ref[0, sl, hi * n:(hi + 1) * n]
            r = st_ref[hi]

            e_pos = epos_t[:, hi:hi + 1]
            e_neg = eneg_t[:, hi:hi + 1]

            b_sc = b * e_neg

            scores = lax.dot_general(
                c, b_sc, (((1,), (1,)), ((), ())),
                preferred_element_type=jnp.float32)
            scores = jnp.where(ltri, scores, 0.0)

            y_diag = jnp.dot(scores, x, preferred_element_type=jnp.float32)
            y_off = lax.dot_general(
                c, r, (((1,), (1,)), ((), ())),
                preferred_element_type=jnp.float32)
            y_ref[0, sl, hi * p:(hi + 1) * p] = e_pos * (y_diag + y_off)

            local = lax.dot_general(
                x, b_sc, (((0,), (0,)), ((), ())),
                preferred_element_type=jnp.float32)
            st_ref[hi] = elast[hi:hi + 1, 0:1] * (r + local)


def kernel(X, initial_states, A, B, C):
    b, S, h, p = X.shape
    n = B.shape[-1]
    nc = S // _LB

    Xf = X.reshape(b, S, h * p)
    Bf = B.reshape(b, S, h * n)
    Cf = C.reshape(b, S, h * n)
    Ir = initial_states.reshape(b, h, p, n)

    Yf = pl.pallas_call(
        _ssd_kernel,
        out_shape=jax.ShapeDtypeStruct((b, S, h * p), jnp.float32),
        grid=(b, nc),
        in_specs=[
            pl.BlockSpec((1, _LB, h * p), lambda i, k: (i, k, 0)),
            pl.BlockSpec((1, _LB, h), lambda i, k: (i, k, 0)),
            pl.BlockSpec((1, _LB, h * n), lambda i, k: (i, k, 0)),
            pl.BlockSpec((1, _LB, h * n), lambda i, k: (i, k, 0)),
            pl.BlockSpec((1, h, p, n), lambda i, k: (i, 0, 0, 0)),
        ],
        out_specs=pl.BlockSpec((1, _LB, h * p), lambda i, k: (i, k, 0)),
        scratch_shapes=[pltpu.VMEM((h, p, n), jnp.float32)],
        compiler_params=pltpu.CompilerParams(
            dimension_semantics=("arbitrary", "arbitrary"),
            vmem_limit_bytes=50 * 1024 * 1024,
        ),
    )(Xf, A, Bf, Cf, Ir)

    return Yf.reshape(b, S, h, p)


# speedup vs baseline: 1.0902x; 1.0902x over previous
import jax
import jax.numpy as jnp
from jax import lax
from jax.experimental import pallas as pl
from jax.experimental.pallas import tpu as pltpu

_L = 256
_LB = 1024


def _ssd_kernel(x_ref, a_ref, b_ref, c_ref, init_ref, y_ref, st_ref):
    k = pl.program_id(1)
    h, p, n = st_ref.shape

    @pl.when(k == 0)
    def _():
        st_ref[...] = init_ref[0]

    row = lax.broadcasted_iota(jnp.int32, (_L, _L), 0)
    col = lax.broadcasted_iota(jnp.int32, (_L, _L), 1)
    ltri = row >= col

    for sub in range(_LB // _L):
        sl = slice(sub * _L, (sub + 1) * _L)
        at = jnp.transpose(a_ref[0, sl, :])

        for hi in range(h):
            x = x_ref[0, sl, hi * p:(hi + 1) * p]
            b = b_ref[0, sl, hi * n:(hi + 1) * n]
            c = c_ref[0, sl, hi * n:(hi + 1) * n]
            a = at[hi:hi + 1, :]
            r = st_ref[hi]

            a_b = jnp.broadcast_to(a, (_L, _L))
            csum = jnp.sum(jnp.where(ltri, a_b, 0.0), axis=1, keepdims=True)
            a_last = jnp.sum(a, axis=1, keepdims=True)
            e_pos = jnp.exp(csum)
            e_neg = jnp.exp(-csum)

            b_sc = b * e_neg

            scores = lax.dot_general(
                c, b_sc, (((1,), (1,)), ((), ())),
                preferred_element_type=jnp.float32)
            scores = jnp.where(ltri, scores, 0.0)

            y_diag = jnp.dot(scores, x, preferred_element_type=jnp.float32)
            y_off = lax.dot_general(
                c, r, (((1,), (1,)), ((), ())),
                preferred_element_type=jnp.float32)
            y_ref[0, sl, hi * p:(hi + 1) * p] = e_pos * (y_diag + y_off)

            local = lax.dot_general(
                x, b_sc, (((0,), (0,)), ((), ())),
                preferred_element_type=jnp.float32)
            st_ref[hi] = jnp.exp(a_last) * (r + local)


def kernel(X, initial_states, A, B, C):
    b, S, h, p = X.shape
    n = B.shape[-1]
    nc = S // _LB

    Xf = X.reshape(b, S, h * p)
    Bf = B.reshape(b, S, h * n)
    Cf = C.reshape(b, S, h * n)
    Ir = initial_states.reshape(b, h, p, n)

    Yf = pl.pallas_call(
        _ssd_kernel,
        out_shape=jax.ShapeDtypeStruct((b, S, h * p), jnp.float32),
        grid=(b, nc),
        in_specs=[
            pl.BlockSpec((1, _LB, h * p), lambda i, k: (i, k, 0)),
            pl.BlockSpec((1, _LB, h), lambda i, k: (i, k, 0)),
            pl.BlockSpec((1, _LB, h * n), lambda i, k: (i, k, 0)),
            pl.BlockSpec((1, _LB, h * n), lambda i, k: (i, k, 0)),
            pl.BlockSpec((1, h, p, n), lambda i, k: (i, 0, 0, 0)),
        ],
        out_specs=pl.BlockSpec((1, _LB, h * p), lambda i, k: (i, k, 0)),
        scratch_shapes=[pltpu.VMEM((h, p, n), jnp.float32)],
        compiler_params=pltpu.CompilerParams(
            dimension_semantics=("arbitrary", "arbitrary"),
            vmem_limit_bytes=50 * 1024 * 1024,
        ),
    )(Xf, A, Bf, Cf, Ir)

    return Yf.reshape(b, S, h, p)
